# Initial kernel scaffold; baseline (speedup 1.0000x reference)
import jax
import jax.numpy as jnp
from jax.experimental import pallas as pl


def kernel(matrix, W_sort_inner, W_dot_inner, W_sort_final, W_dot_final):
    raise NotImplementedError("write your pallas kernel here")


# speedup vs baseline: 17.3086x; 17.3086x over previous
import numpy as np
import jax
import jax.numpy as jnp
from jax.experimental import pallas as pl

_DIM = 2
_N = 32
_P = _N * (_N - 1)
_L = 2 * _DIM * _N + 1
_D = _DIM * _DIM + _L
_PPAD = 1024
_PC = 128
_NQ = _PPAD // _PC
_BIG = float(np.float32(1e30))


def _selectors():
    import itertools as it
    perms = np.array(list(it.permutations(range(_N), _DIM)), dtype=np.int32)
    p0 = np.zeros((_N, _PPAD), dtype=np.float32)
    p1 = np.zeros((_N, _PPAD), dtype=np.float32)
    cols = np.arange(_P)
    p0[perms[:, 0], cols] = 1.0
    p1[perms[:, 1], cols] = 1.0
    return p0, p1


_P0, _P1 = _selectors()


def _oddeven_pairs(n):
    pairs = []
    p = 1
    while p < n:
        k = p
        while k >= 1:
            for j in range(k % p, n - k, 2 * k):
                for i in range(min(k, n - j - k)):
                    if (i + j) // (p * 2) == (i + j + k) // (p * 2):
                        pairs.append((i + j, i + j + k))
            k //= 2
        p *= 2
    return pairs


_OE32 = _oddeven_pairs(_N)


def _chunk_scores(G, p0, p1, ws0, ws1, wdP, wsf):
    hi_prec = jax.lax.Precision.HIGHEST
    xt = jnp.dot(G, p0, precision=hi_prec)
    yt = jnp.dot(G, p1, precision=hi_prec)

    g00 = jnp.sum(xt * p0, axis=0, keepdims=True)
    g01 = jnp.sum(xt * p1, axis=0, keepdims=True)
    g11 = jnp.sum(yt * p1, axis=0, keepdims=True)

    planes = []
    for k in range(_N):
        xk = xt[k:k + 1, :]
        yk = yt[k:k + 1, :]
        t = ws0 * xk + ws1 * yk
        hit = (p0[k:k + 1, :] + p1[k:k + 1, :]) > 0.5
        planes.append(jnp.where(hit, _BIG, t))

    for a, b in _OE32:
        pa, pb = planes[a], planes[b]
        planes[a] = jnp.minimum(pa, pb)
        planes[b] = jnp.maximum(pa, pb)

    acc = planes[0] * wdP[:, 0:1]
    for k in range(1, _N):
        acc = acc + planes[k] * wdP[:, k:k + 1]

    vecT = jnp.concatenate([g00, g01, g01, g11, acc], axis=0)
    return jnp.dot(wsf, vecT, precision=hi_prec)


def _body(mT_ref, m_ref, ws0_ref, ws1_ref, wdP_ref, wsf_ref,
          p0_ref, p1_ref, wdfP_ref, out_ref):
    hi_prec = jax.lax.Precision.HIGHEST
    G = jnp.dot(mT_ref[...], m_ref[...], precision=hi_prec)
    ws0 = ws0_ref[...]
    ws1 = ws1_ref[...]
    wdP = wdP_ref[...]
    wsf = wsf_ref[...]

    ir = jax.lax.broadcasted_iota(jnp.int32, (1, _PC), 1)

    sp = []
    for q in range(_NQ):
        s = _chunk_scores(G, p0_ref[:, q * _PC:(q + 1) * _PC],
                          p1_ref[:, q * _PC:(q + 1) * _PC],
                          ws0, ws1, wdP, wsf)
        if q == _NQ - 1:
            s = jnp.where(ir >= _PC - (_PPAD - _P), _BIG, s)
        sp.append(s)

    k = 2
    while k <= _PPAD:
        j = k // 2
        while j >= 1:
            if j < _NQ:
                for q in range(_NQ):
                    c = q ^ j
                    if c <= q:
                        continue
                    A, B = sp[q], sp[c]
                    lo = jnp.minimum(A, B)
                    hi = jnp.maximum(A, B)
                    if k < _NQ:
                        if (q & k) == 0:
                            sp[q], sp[c] = lo, hi
                        else:
                            sp[q], sp[c] = hi, lo
                    else:
                        lm = (ir & (k >> 3)) == 0
                        sp[q] = jnp.where(lm, lo, hi)
                        sp[c] = jnp.where(lm, hi, lo)
            else:
                j2 = j >> 3
                bj = (ir & j2) == 0
                tl = bj == ((ir & (k >> 3)) == 0)
                for q in range(_NQ):
                    x = sp[q]
                    rl = jnp.concatenate([x[:, j2:], x[:, :j2]], axis=1)
                    rr = jnp.concatenate([x[:, -j2:], x[:, :-j2]], axis=1)
                    partner = jnp.where(bj, rl, rr)
                    lo = jnp.minimum(x, partner)
                    hi = jnp.maximum(x, partner)
                    sp[q] = jnp.where(tl, lo, hi)
            j //= 2
        k *= 2

    acc = sp[0] * wdfP_ref[0]
    for q in range(1, _NQ):
        acc = acc + sp[q] * wdfP_ref[q]
    out_ref[...] = jnp.sum(acc, axis=1, keepdims=True)


def kernel(matrix, W_sort_inner, W_dot_inner, W_sort_final, W_dot_final):
    f32 = jnp.float32
    matrix = matrix.astype(f32)
    mT = matrix.T
    ws0 = W_sort_inner[:, 0:1].astype(f32)
    ws1 = W_sort_inner[:, 1:2].astype(f32)
    wdP = jnp.pad(W_dot_inner.astype(f32), ((0, 0), (0, _DIM)))
    wsf = W_sort_final.astype(f32)
    wdf_pad = jnp.pad(W_dot_final.astype(f32), ((0, 0), (0, _PPAD - _P)))
    wdfP = jnp.transpose(wdf_pad.reshape(_L, _PC, _NQ), (2, 0, 1))

    p0 = jnp.asarray(_P0)
    p1 = jnp.asarray(_P1)

    out = pl.pallas_call(
        _body,
        out_shape=jax.ShapeDtypeStruct((_L, 1), f32),
    )(mT, matrix, ws0, ws1, wdP, wsf, p0, p1, wdfP)
    return out.reshape(_L)


# speedup vs baseline: 18.2446x; 1.0541x over previous
import numpy as np
import jax
import jax.numpy as jnp
from jax.experimental import pallas as pl

_DIM = 2
_N = 32
_P = _N * (_N - 1)
_L = 2 * _DIM * _N + 1
_D = _DIM * _DIM + _L
_PPAD = 1024
_PC = 128
_NQ = _PPAD // _PC
_BIG = float(np.float32(1e30))


def _selectors():
    import itertools as it
    perms = np.array(list(it.permutations(range(_N), _DIM)), dtype=np.int32)
    p0 = np.zeros((_N, _PPAD), dtype=np.float32)
    p1 = np.zeros((_N, _PPAD), dtype=np.float32)
    cols = np.arange(_P)
    p0[perms[:, 0], cols] = 1.0
    p1[perms[:, 1], cols] = 1.0
    return p0, p1


_P0, _P1 = _selectors()


def _oddeven_pairs(n):
    pairs = []
    p = 1
    while p < n:
        k = p
        while k >= 1:
            for j in range(k % p, n - k, 2 * k):
                for i in range(min(k, n - j - k)):
                    if (i + j) // (p * 2) == (i + j + k) // (p * 2):
                        pairs.append((i + j, i + j + k))
            k //= 2
        p *= 2
    return pairs


_OE32 = _oddeven_pairs(_N)


def _chunk_scores(G, p0, p1, ws0, ws1, wdP, wsf):
    hi_prec = jax.lax.Precision.HIGHEST
    xt = jnp.dot(G, p0)
    yt = jnp.dot(G, p1)

    g00 = jnp.sum(xt * p0, axis=0, keepdims=True)
    g01 = jnp.sum(xt * p1, axis=0, keepdims=True)
    g11 = jnp.sum(yt * p1, axis=0, keepdims=True)

    planes = []
    for k in range(_N):
        xk = xt[k:k + 1, :]
        yk = yt[k:k + 1, :]
        t = ws0 * xk + ws1 * yk
        hit = (p0[k:k + 1, :] + p1[k:k + 1, :]) > 0.5
        planes.append(jnp.where(hit, _BIG, t))

    for a, b in _OE32:
        pa, pb = planes[a], planes[b]
        planes[a] = jnp.minimum(pa, pb)
        planes[b] = jnp.maximum(pa, pb)

    acc = planes[0] * wdP[:, 0:1]
    for k in range(1, _N - _DIM):
        acc = acc + planes[k] * wdP[:, k:k + 1]

    vecT = jnp.concatenate([g00, g01, g01, g11, acc], axis=0)
    return jnp.dot(wsf, vecT, precision=hi_prec)


def _body(mT_ref, m_ref, wsi_ref, wdi_ref, wsf_ref,
          p0_ref, p1_ref, wdfP_ref, out_ref):
    hi_prec = jax.lax.Precision.HIGHEST
    G = jnp.dot(mT_ref[...], m_ref[...], precision=hi_prec)
    ws0 = wsi_ref[:, 0:1]
    ws1 = wsi_ref[:, 1:2]
    wdP = wdi_ref[...]
    wsf = wsf_ref[...]

    ir = jax.lax.broadcasted_iota(jnp.int32, (1, _PC), 1)

    sp = []
    for q in range(_NQ):
        s = _chunk_scores(G, p0_ref[:, q * _PC:(q + 1) * _PC],
                          p1_ref[:, q * _PC:(q + 1) * _PC],
                          ws0, ws1, wdP, wsf)
        if q == _NQ - 1:
            s = jnp.where(ir >= _PC - (_PPAD - _P), _BIG, s)
        sp.append(s)

    k = 2
    while k <= _PPAD:
        j = k // 2
        while j >= 1:
            if j < _NQ:
                for q in range(_NQ):
                    c = q ^ j
                    if c <= q:
                        continue
                    A, B = sp[q], sp[c]
                    lo = jnp.minimum(A, B)
                    hi = jnp.maximum(A, B)
                    if k < _NQ:
                        if (q & k) == 0:
                            sp[q], sp[c] = lo, hi
                        else:
                            sp[q], sp[c] = hi, lo
                    else:
                        lm = (ir & (k >> 3)) == 0
                        sp[q] = jnp.where(lm, lo, hi)
                        sp[c] = jnp.where(lm, hi, lo)
            else:
                j2 = j >> 3
                bj = (ir & j2) == 0
                tl = bj == ((ir & (k >> 3)) == 0)
                for q in range(_NQ):
                    x = sp[q]
                    rl = jnp.concatenate([x[:, j2:], x[:, :j2]], axis=1)
                    rr = jnp.concatenate([x[:, -j2:], x[:, :-j2]], axis=1)
                    partner = jnp.where(bj, rl, rr)
                    lo = jnp.minimum(x, partner)
                    hi = jnp.maximum(x, partner)
                    sp[q] = jnp.where(tl, lo, hi)
            j //= 2
        k *= 2

    acc = sp[0] * wdfP_ref[0]
    for q in range(1, _NQ):
        acc = acc + sp[q] * wdfP_ref[q]
    out_ref[...] = jnp.sum(acc, axis=1, keepdims=True)


def kernel(matrix, W_sort_inner, W_dot_inner, W_sort_final, W_dot_final):
    f32 = jnp.float32
    matrix = matrix.astype(f32)
    mT = matrix.T
    wsi = W_sort_inner.astype(f32)
    wdi = W_dot_inner.astype(f32)
    wsf = W_sort_final.astype(f32)
    wdf_pad = jnp.pad(W_dot_final.astype(f32), ((0, 0), (0, _PPAD - _P)))
    wdfP = jnp.transpose(wdf_pad.reshape(_L, _PC, _NQ), (2, 0, 1))

    p0 = jnp.asarray(_P0)
    p1 = jnp.asarray(_P1)

    out = pl.pallas_call(
        _body,
        out_shape=jax.ShapeDtypeStruct((_L, 1), f32),
    )(mT, matrix, wsi, wdi, wsf, p0, p1, wdfP)
    return out.reshape(_L)


# speedup vs baseline: 20.5354x; 1.1256x over previous
import numpy as np
import jax
import jax.numpy as jnp
from jax.experimental import pallas as pl

_DIM = 2
_N = 32
_P = _N * (_N - 1)
_L = 2 * _DIM * _N + 1
_D = _DIM * _DIM + _L
_PPAD = 1024
_PC = 128
_NQ = _PPAD // _PC
_BIG = float(np.float32(1e30))


def _selectors():
    import itertools as it
    perms = np.array(list(it.permutations(range(_N), _DIM)), dtype=np.int32)
    p0 = np.zeros((_N, _PPAD), dtype=np.float32)
    p1 = np.zeros((_N, _PPAD), dtype=np.float32)
    cols = np.arange(_P)
    p0[perms[:, 0], cols] = 1.0
    p1[perms[:, 1], cols] = 1.0
    return p0, p1


_P0, _P1 = _selectors()


def _oddeven_pairs(n):
    pairs = []
    p = 1
    while p < n:
        k = p
        while k >= 1:
            for j in range(k % p, n - k, 2 * k):
                for i in range(min(k, n - j - k)):
                    if (i + j) // (p * 2) == (i + j + k) // (p * 2):
                        pairs.append((i + j, i + j + k))
            k //= 2
        p *= 2
    return pairs


_OE32 = _oddeven_pairs(_N)


def _chunk_scores(G, p0, p1, ws0, ws1, wdP, wsf):
    hi_prec = jax.lax.Precision.HIGHEST
    xt = jnp.dot(G, p0)
    yt = jnp.dot(G, p1)

    g00 = jnp.sum(xt * p0, axis=0, keepdims=True)
    g01 = jnp.sum(xt * p1, axis=0, keepdims=True)
    g11 = jnp.sum(yt * p1, axis=0, keepdims=True)

    planes = []
    for k in range(_N):
        xk = xt[k:k + 1, :]
        yk = yt[k:k + 1, :]
        t = ws0 * xk + ws1 * yk
        hit = (p0[k:k + 1, :] + p1[k:k + 1, :]) > 0.5
        planes.append(jnp.where(hit, _BIG, t))

    for a, b in _OE32:
        pa, pb = planes[a], planes[b]
        planes[a] = jnp.minimum(pa, pb)
        planes[b] = jnp.maximum(pa, pb)

    acc = planes[0] * wdP[:, 0:1]
    for k in range(1, _N - _DIM):
        acc = acc + planes[k] * wdP[:, k:k + 1]

    vecT = jnp.concatenate([g00, g01, g01, g11, acc], axis=0)
    return jnp.dot(wsf, vecT, precision=hi_prec)


def _body(mT_ref, m_ref, wsi_ref, wdi_ref, wsf_ref,
          p0_ref, p1_ref, wdfP_ref, out_ref):
    hi_prec = jax.lax.Precision.HIGHEST
    G = jnp.dot(mT_ref[...], m_ref[...], precision=hi_prec)
    ws0 = wsi_ref[:, 0:1]
    ws1 = wsi_ref[:, 1:2]
    wdP = wdi_ref[...]
    wsf = wsf_ref[...]

    sp = []
    for q in range(_NQ):
        s = _chunk_scores(G, p0_ref[:, q * _PC:(q + 1) * _PC],
                          p1_ref[:, q * _PC:(q + 1) * _PC],
                          ws0, ws1, wdP, wsf)
        sT = jnp.transpose(s, (1, 0))
        for gidx in range(4):
            sp.append(sT[gidx * _N:(gidx + 1) * _N, :])

    sp[31] = jnp.full((_N, _L), _BIG, jnp.float32)

    ia = jax.lax.broadcasted_iota(jnp.int32, (_N, 1), 0)
    k = 2
    while k <= _PPAD:
        j = k // 2
        while j >= 1:
            if j < _N:
                for b in range(_N):
                    c = b ^ j
                    if c <= b:
                        continue
                    A, B = sp[b], sp[c]
                    lo = jnp.minimum(A, B)
                    hi = jnp.maximum(A, B)
                    if k < _N:
                        if (b & k) == 0:
                            sp[b], sp[c] = lo, hi
                        else:
                            sp[b], sp[c] = hi, lo
                    else:
                        am = (ia & (k >> 5)) == 0
                        sp[b] = jnp.where(am, lo, hi)
                        sp[c] = jnp.where(am, hi, lo)
            else:
                j2 = j >> 5
                bj = (ia & j2) == 0
                tl = bj == ((ia & (k >> 5)) == 0)
                for b in range(_N):
                    x = sp[b]
                    rl = jnp.concatenate([x[j2:], x[:j2]], axis=0)
                    rr = jnp.concatenate([x[-j2:], x[:-j2]], axis=0)
                    partner = jnp.where(bj, rl, rr)
                    lo = jnp.minimum(x, partner)
                    hi = jnp.maximum(x, partner)
                    sp[b] = jnp.where(tl, lo, hi)
            j //= 2
        k *= 2

    acc = sp[0] * wdfP_ref[0]
    for b in range(1, _N):
        acc = acc + sp[b] * wdfP_ref[b]
    out_ref[...] = jnp.sum(acc, axis=0, keepdims=True)


def kernel(matrix, W_sort_inner, W_dot_inner, W_sort_final, W_dot_final):
    f32 = jnp.float32
    matrix = matrix.astype(f32)
    mT = matrix.T
    wsi = W_sort_inner.astype(f32)
    wdi = W_dot_inner.astype(f32)
    wsf = W_sort_final.astype(f32)
    wdf_pad = jnp.pad(W_dot_final.astype(f32), ((0, 0), (0, _PPAD - _P)))
    wdfP = jnp.transpose(wdf_pad.T.reshape(_N, _N, _L), (1, 0, 2))

    p0 = jnp.asarray(_P0)
    p1 = jnp.asarray(_P1)

    out = pl.pallas_call(
        _body,
        out_shape=jax.ShapeDtypeStruct((1, _L), f32),
    )(mT, matrix, wsi, wdi, wsf, p0, p1, wdfP)
    return out.reshape(_L)
